# Initial kernel scaffold; baseline (speedup 1.0000x reference)
import jax
import jax.numpy as jnp
from jax.experimental import pallas as pl


def kernel(rois_all):
    raise NotImplementedError("write your pallas kernel here")


# speedup vs baseline: 4.9353x; 4.9353x over previous
import jax
import jax.numpy as jnp
from jax import lax
from jax.experimental import pallas as pl
from jax.experimental.pallas import tpu as pltpu
from jax.experimental.pallas import tpu_sc as plsc

B = 8
N = 20000
K = 200
C = 5
L = 16
KPAD = 208
NITER = N // L
BLK_STEPS = 25
SENT = B * N
EPAD = KPAD * 4
ECH = EPAD // 8


def _body(cls_hbm, rois_hbm, out_hbm, cls_v, idx_v, idx2_v, rows_v, cnt_ref, sem):
    cid = lax.axis_index("c")
    sid = lax.axis_index("s")
    w = sid * 2 + cid
    b = w % B

    pltpu.sync_copy(cls_hbm.at[b], cls_v)
    iota = lax.iota(jnp.int32, L)
    base = b * N

    def run_task(c):
        for kk in range(KPAD // L):
            idx_v[pl.ds(kk * L, L)] = jnp.full((L,), SENT, jnp.int32)
        cnt_ref[0] = jnp.int32(0)

        def outer(blk, carry):
            @pl.when(cnt_ref[0] < K)
            def _():
                def inner(j, cnt):
                    i = blk * BLK_STEPS + j
                    v = cls_v[pl.ds(i * L, L)]
                    m = v == c
                    inc = jnp.where(m, 1, 0).astype(jnp.int32)
                    csum = plsc.cumsum(inc)
                    pos = cnt + csum - 1
                    ok = jnp.logical_and(m, pos < K)
                    plsc.store_scatter(
                        idx_v, [pos], base + i * L + iota, mask=ok)
                    return cnt + csum[L - 1]

                cnt_ref[0] = lax.fori_loop(0, BLK_STEPS, inner, cnt_ref[0])
            return carry

        lax.fori_loop(0, NITER // BLK_STEPS, outer, jnp.int32(0))

        for q in range(EPAD // L):
            e = q * L + iota
            row = plsc.load_gather(idx_v, [e >> 2])
            idx2_v[pl.ds(q * L, L)] = row * 6 + 2 + (e & 3)

        cps = []
        for h in range(8):
            cps.append(pltpu.async_copy(
                rois_hbm.at[idx2_v.at[pl.ds(h * ECH, ECH)]],
                rows_v.at[pl.ds(h * ECH, ECH)], sem))
        for cp in cps:
            cp.wait()

        cnt4 = cnt_ref[0] * 4
        for q in range(EPAD // L):
            e = q * L + iota
            val = rows_v[pl.ds(q * L, L)]
            rows_v[pl.ds(q * L, L)] = jnp.where(e < cnt4, val, 0.0)

        t = (c - 1) * B + b
        pltpu.sync_copy(rows_v.at[pl.ds(0, K * 4)], out_hbm.at[pl.ds(t * K * 4, K * 4)])

    run_task(w // B + 1)

    @pl.when(w < B)
    def _():
        run_task(jnp.int32(C))


def kernel(rois_all):
    cls = rois_all[:, :, 0].astype(jnp.int32)
    roisflat = rois_all.reshape(B * N * 6)

    mesh = plsc.VectorSubcoreMesh(
        core_axis_name="c", subcore_axis_name="s", num_cores=2, num_subcores=16)
    out = pl.kernel(
        _body,
        out_type=jax.ShapeDtypeStruct((C * B * K * 4,), jnp.float32),
        mesh=mesh,
        compiler_params=pltpu.CompilerParams(needs_layout_passes=False),
        scratch_types=[
            pltpu.VMEM((N,), jnp.int32),
            pltpu.VMEM((KPAD,), jnp.int32),
            pltpu.VMEM((EPAD,), jnp.int32),
            pltpu.VMEM((EPAD,), jnp.float32),
            pltpu.SMEM((1,), jnp.int32),
            pltpu.SemaphoreType.DMA,
        ],
    )(cls, roisflat)

    o = out.reshape(C, B, K, 4)
    return tuple(o[i] for i in range(C))
